# baseline (speedup 1.0000x reference)
import jax
import jax.numpy as jnp
from jax.experimental import pallas as pl
from jax.experimental.pallas import tpu as pltpu

_NCHUNK = 8


def _sdhgcn_body(adj_hbm, x_ref, w_ref, out_ref, abuf, sems):
    n = out_ref.shape[0]
    ck = n // _NCHUNK
    copies = []
    for i in range(_NCHUNK):
        c = pltpu.make_async_copy(
            adj_hbm.at[pl.ds(i * ck, ck), :], abuf.at[i], sems.at[i])
        c.start()
        copies.append(c)

    xw = jnp.dot(x_ref[...], w_ref[...],
                 preferred_element_type=jnp.float32)
    acc = None
    deg = None
    for i in range(_NCHUNK):
        copies[i].wait()
        a = abuf[i].astype(jnp.float32)
        part = jax.lax.dot_general(
            a, xw[i * ck:(i + 1) * ck, :],
            dimension_numbers=(((0,), (0,)), ((), ())),
            preferred_element_type=jnp.float32)
        dpart = jnp.sum(a, axis=0)
        acc = part if acc is None else acc + part
        deg = dpart if deg is None else deg + dpart

    coeff = jax.lax.rsqrt(jnp.maximum(deg, 1.0))
    out_ref[...] = jnp.maximum(acc * coeff[:, None], 0.0)


def kernel(X, adj_matrix, weight):
    n, d_in = X.shape
    d_out = weight.shape[1]
    return pl.pallas_call(
        _sdhgcn_body,
        in_specs=[
            pl.BlockSpec(memory_space=pl.ANY),
            pl.BlockSpec(memory_space=pltpu.VMEM),
            pl.BlockSpec(memory_space=pltpu.VMEM),
        ],
        out_specs=pl.BlockSpec(memory_space=pltpu.VMEM),
        out_shape=jax.ShapeDtypeStruct((n, d_out), jnp.float32),
        scratch_shapes=[
            pltpu.VMEM((_NCHUNK, n // _NCHUNK, n), jnp.int32),
            pltpu.SemaphoreType.DMA((_NCHUNK,)),
        ],
    )(adj_matrix, X, weight)


# speedup vs baseline: 1.3291x; 1.3291x over previous
import jax
import jax.numpy as jnp
from jax.experimental import pallas as pl


def _sdhgcn_body(adj_ref, x_ref, w_ref, out_ref):
    a = adj_ref[...].astype(jnp.float32)
    xw = jnp.dot(x_ref[...], w_ref[...],
                 preferred_element_type=jnp.float32)
    out_t = jax.lax.dot_general(
        xw, a, dimension_numbers=(((0,), (0,)), ((), ())),
        preferred_element_type=jnp.float32)
    deg = jnp.sum(a, axis=0)
    coeff = jax.lax.rsqrt(jnp.maximum(deg, 1.0))
    out_ref[...] = jnp.maximum(out_t * coeff[None, :], 0.0).T


def kernel(X, adj_matrix, weight):
    n, d_out = X.shape[0], weight.shape[1]
    return pl.pallas_call(
        _sdhgcn_body,
        out_shape=jax.ShapeDtypeStruct((n, d_out), jnp.float32),
    )(adj_matrix, X, weight)
